# baseline (speedup 1.0000x reference)
import jax
import jax.numpy as jnp
from jax.experimental import pallas as pl

D_MODEL_K = 2048
K_HALF = D_MODEL_K // 2
N_EXP = 64
BLOCK_T = 2048


def _router_body(x1_ref, x2_ref, w_ref, idx_ref, p1_ref, prob_ref):
    i = pl.program_id(0)
    w = w_ref[...]
    st = jax.lax.dot_general(
        w[:, :K_HALF], x1_ref[...], (((1,), (1,)), ((), ())),
        preferred_element_type=jnp.float32)
    st = st + jax.lax.dot_general(
        w[:, K_HALF:], x2_ref[...], (((1,), (1,)), ((), ())),
        preferred_element_type=jnp.float32)
    m = jnp.max(st, axis=0, keepdims=True)
    e = jnp.exp(st - m)
    denom = jnp.sum(e, axis=0, keepdims=True)
    r = 1.0 / denom
    prob_ref[...] = (e * r).T
    ii = jax.lax.broadcasted_iota(jnp.int32, st.shape, 0)
    idx_ref[pl.ds(i, 1), :] = jnp.min(
        jnp.where(st == m, ii, N_EXP), axis=0, keepdims=True)
    p1_ref[pl.ds(i, 1), :] = r


def kernel(x, W):
    n_tok = x.shape[0]
    g = n_tok // BLOCK_T
    out_shapes = (
        jax.ShapeDtypeStruct((g, BLOCK_T), jnp.int32),
        jax.ShapeDtypeStruct((g, BLOCK_T), jnp.float32),
        jax.ShapeDtypeStruct((n_tok, N_EXP), jnp.float32),
    )
    idx, p1, prob = pl.pallas_call(
        _router_body,
        grid=(g,),
        in_specs=[
            pl.BlockSpec((BLOCK_T, K_HALF), lambda i: (i, 0)),
            pl.BlockSpec((BLOCK_T, K_HALF), lambda i: (i, 1)),
            pl.BlockSpec((N_EXP, D_MODEL_K), lambda i: (0, 0)),
        ],
        out_specs=(
            pl.BlockSpec((g, BLOCK_T), lambda i: (0, 0)),
            pl.BlockSpec((g, BLOCK_T), lambda i: (0, 0)),
            pl.BlockSpec((BLOCK_T, N_EXP), lambda i: (i, 0)),
        ),
        out_shape=out_shapes,
    )(x, x, W)
    return (idx.reshape(n_tok), p1.reshape(n_tok), prob)


# speedup vs baseline: 1.0581x; 1.0581x over previous
import jax
import jax.numpy as jnp
from jax.experimental import pallas as pl

D_MODEL_K = 2048
N_EXP = 64
BLOCK_T = 2048


def _router_body(x_ref, w_ref, idx_ref, p1_ref, prob_ref):
    st = jax.lax.dot_general(
        w_ref[...], x_ref[...], (((1,), (1,)), ((), ())),
        preferred_element_type=jnp.float32)
    m = jnp.max(st, axis=0, keepdims=True)
    e = jnp.exp(st - m)
    denom = jnp.sum(e, axis=0, keepdims=True)
    r = 1.0 / denom
    prob_ref[...] = (e * r).T
    ii = jax.lax.broadcasted_iota(jnp.int32, st.shape, 0)
    idx_ref[0] = jnp.min(jnp.where(st == m, ii, N_EXP), axis=0, keepdims=True)
    p1_ref[0] = r


def kernel(x, W):
    n_tok = x.shape[0]
    g = n_tok // BLOCK_T
    out_shapes = (
        jax.ShapeDtypeStruct((g, 1, BLOCK_T), jnp.int32),
        jax.ShapeDtypeStruct((g, 1, BLOCK_T), jnp.float32),
        jax.ShapeDtypeStruct((n_tok, N_EXP), jnp.float32),
    )
    idx, p1, prob = pl.pallas_call(
        _router_body,
        grid=(g,),
        in_specs=[
            pl.BlockSpec((BLOCK_T, D_MODEL_K), lambda i: (i, 0)),
            pl.BlockSpec((N_EXP, D_MODEL_K), lambda i: (0, 0)),
        ],
        out_specs=(
            pl.BlockSpec((1, 1, BLOCK_T), lambda i: (i, 0, 0)),
            pl.BlockSpec((1, 1, BLOCK_T), lambda i: (i, 0, 0)),
            pl.BlockSpec((BLOCK_T, N_EXP), lambda i: (i, 0)),
        ),
        out_shape=out_shapes,
    )(x, W)
    return (idx.reshape(n_tok), p1.reshape(n_tok), prob)
